# baseline (speedup 1.0000x reference)
import functools

import jax
import jax.numpy as jnp
from jax import lax
from jax.experimental import pallas as pl
from jax.experimental.pallas import tpu as pltpu
from jax.experimental.pallas import tpu_sc as plsc

N = 10000
E = 160000
D = 256
DH = 128
NC = 2
NS = 16
ROWS_MAIN = 632
ROWS_LAST = 520
BLK = 400
NBLK = N // BLK

DEG_EDGES_PER_W = E // (NC * NS)
DEG_CHUNK = 40
DEG_NCHUNK = DEG_EDGES_PER_W // DEG_CHUNK
DEG_NBUF = 5
DEG_OUTER = DEG_NCHUNK // DEG_NBUF
DEG_W = 128
DEG_WOUT = 16

PROP_EDGES_PER_SUB = E // NS
PROP_CHUNK = 40
PROP_NCHUNK = PROP_EDGES_PER_SUB // PROP_CHUNK
PROP_NBUF = 5
PROP_OUTER = PROP_NCHUNK // PROP_NBUF
PROP_EPI = PROP_NCHUNK - PROP_OUTER * PROP_NBUF

_MESH = plsc.VectorSubcoreMesh(core_axis_name="c", subcore_axis_name="s")


@functools.partial(
    pl.kernel,
    out_type=jax.ShapeDtypeStruct((NC, N, DEG_W), jnp.float32),
    mesh=_MESH,
    scratch_types=[
        pltpu.VMEM((DEG_EDGES_PER_W,), jnp.int32),
        pltpu.VMEM((DEG_CHUNK, DEG_W), jnp.float32),
        pltpu.VMEM_SHARED((N, DEG_W), jnp.float32),
        pltpu.SemaphoreType.DMA,
    ],
)
def _deg_kernel(dst_hbm, zeros_hbm, ones_hbm, degp_hbm, dst_all, ones_v,
                acc_sh, sem):
    c = lax.axis_index("c")
    s = lax.axis_index("s")

    @pl.when(s < NS - 1)
    def _():
        pltpu.sync_copy(zeros_hbm.at[pl.ds(0, ROWS_MAIN), :],
                        acc_sh.at[pl.ds(s * ROWS_MAIN, ROWS_MAIN), :])

    @pl.when(s == NS - 1)
    def _():
        pltpu.sync_copy(zeros_hbm.at[pl.ds(0, ROWS_LAST), :],
                        acc_sh.at[pl.ds((NS - 1) * ROWS_MAIN, ROWS_LAST), :])

    pltpu.sync_copy(ones_hbm, ones_v)
    w = c * NS + s
    pltpu.sync_copy(dst_hbm.at[pl.ds(w * DEG_EDGES_PER_W, DEG_EDGES_PER_W)],
                    dst_all)
    plsc.subcore_barrier()

    def body(t, carry):
        j0 = t * DEG_NBUF
        for b in range(DEG_NBUF):
            @pl.when(t > 0)
            def _():
                pltpu.make_async_copy(zeros_hbm.at[pl.ds(0, DEG_CHUNK), :],
                                      ones_v, sem).wait()
            idx = dst_all.at[pl.ds((j0 + b) * DEG_CHUNK, DEG_CHUNK)]
            pltpu.async_copy(ones_v, acc_sh.at[idx], sem, add=True)
        return carry

    lax.fori_loop(0, DEG_OUTER, body, 0)
    for b in range(DEG_NBUF):
        pltpu.make_async_copy(zeros_hbm.at[pl.ds(0, DEG_CHUNK), :],
                              ones_v, sem).wait()
    plsc.subcore_barrier()

    @pl.when(s < NS - 1)
    def _():
        pltpu.sync_copy(acc_sh.at[pl.ds(s * ROWS_MAIN, ROWS_MAIN), :],
                        degp_hbm.at[c, pl.ds(s * ROWS_MAIN, ROWS_MAIN), :])

    @pl.when(s == NS - 1)
    def _():
        pltpu.sync_copy(acc_sh.at[pl.ds((NS - 1) * ROWS_MAIN, ROWS_LAST), :],
                        degp_hbm.at[c, pl.ds((NS - 1) * ROWS_MAIN, ROWS_LAST), :])


@functools.partial(
    pl.kernel,
    out_type=jax.ShapeDtypeStruct((NC * N, DH), jnp.float32),
    mesh=_MESH,
    scratch_types=[
        pltpu.VMEM((PROP_EDGES_PER_SUB,), jnp.int32),
        pltpu.VMEM((PROP_EDGES_PER_SUB,), jnp.int32),
        [pltpu.VMEM((PROP_CHUNK, DH), jnp.float32) for _ in range(PROP_NBUF)],
        pltpu.VMEM_SHARED((N, DH), jnp.float32),
        [pltpu.SemaphoreType.DMA for _ in range(PROP_NBUF)],
        [pltpu.SemaphoreType.DMA for _ in range(PROP_NBUF)],
    ],
)
def _prop_kernel(xs_hbm, src2_hbm, dst_hbm, zeros_hbm, raw_hbm,
                 src_all, dst_all, bufs, acc_sh, gsems, ssems):
    c = lax.axis_index("c")
    s = lax.axis_index("s")

    @pl.when(s < NS - 1)
    def _():
        pltpu.sync_copy(zeros_hbm.at[pl.ds(0, ROWS_MAIN), :],
                        acc_sh.at[pl.ds(s * ROWS_MAIN, ROWS_MAIN), :])

    @pl.when(s == NS - 1)
    def _():
        pltpu.sync_copy(zeros_hbm.at[pl.ds(0, ROWS_LAST), :],
                        acc_sh.at[pl.ds((NS - 1) * ROWS_MAIN, ROWS_LAST), :])

    pltpu.sync_copy(src2_hbm.at[pl.ds(c * E + s * PROP_EDGES_PER_SUB,
                                      PROP_EDGES_PER_SUB)], src_all)
    pltpu.sync_copy(dst_hbm.at[pl.ds(s * PROP_EDGES_PER_SUB,
                                     PROP_EDGES_PER_SUB)], dst_all)
    plsc.subcore_barrier()

    def body(t, carry):
        j0 = t * PROP_NBUF
        gd = []
        for b in range(PROP_NBUF):
            @pl.when(t > 0)
            def _(b=b):
                pltpu.make_async_copy(
                    zeros_hbm.at[pl.ds(0, PROP_CHUNK), :], bufs[b],
                    ssems[b]).wait()
            idx = src_all.at[pl.ds((j0 + b) * PROP_CHUNK, PROP_CHUNK)]
            gd.append(pltpu.async_copy(xs_hbm.at[idx], bufs[b], gsems[b]))
        for b in range(PROP_NBUF):
            gd[b].wait()
            idx = dst_all.at[pl.ds((j0 + b) * PROP_CHUNK, PROP_CHUNK)]
            pltpu.async_copy(bufs[b], acc_sh.at[idx], ssems[b], add=True)
        return carry

    lax.fori_loop(0, PROP_OUTER, body, 0)
    for b in range(PROP_NBUF):
        pltpu.make_async_copy(zeros_hbm.at[pl.ds(0, PROP_CHUNK), :], bufs[b],
                              ssems[b]).wait()
    plsc.subcore_barrier()

    @pl.when(s < NS - 1)
    def _():
        pltpu.sync_copy(acc_sh.at[pl.ds(s * ROWS_MAIN, ROWS_MAIN), :],
                        raw_hbm.at[pl.ds(c * N + s * ROWS_MAIN, ROWS_MAIN), :])

    @pl.when(s == NS - 1)
    def _():
        pltpu.sync_copy(
            acc_sh.at[pl.ds((NS - 1) * ROWS_MAIN, ROWS_LAST), :],
            raw_hbm.at[pl.ds(c * N + (NS - 1) * ROWS_MAIN, ROWS_LAST), :])


def _dis_from_degp(degp_blk):
    dv = degp_blk[0] + degp_blk[1]
    deg = dv[:, 0:1]
    return jnp.where(deg > 0, lax.rsqrt(deg), 0.0)


def _scale_kernel_body(degp_ref, x_ref, xs_ref):
    dis = _dis_from_degp(degp_ref[...])
    xs = x_ref[...] * dis
    xs_ref[0] = xs[:, :DH]
    xs_ref[1] = xs[:, DH:]


def _arma_h(raw0_ref, raw1_ref, xin_ref, degp_ref, w_ref, b_ref):
    dis = _dis_from_degp(degp_ref[...])
    agg = jnp.concatenate([raw0_ref[...], raw1_ref[...]], axis=1) * dis
    cat = jnp.concatenate([agg, xin_ref[...]], axis=1)
    z = jnp.dot(cat.astype(jnp.bfloat16), w_ref[...],
                preferred_element_type=jnp.float32)
    z = z + b_ref[...]
    h = 0.5 * (jnp.maximum(z[:, :D], 0.0) + jnp.maximum(z[:, D:], 0.0))
    return h, dis


def _layer1_body(raw0_ref, raw1_ref, xin_ref, degp_ref, w_ref, b_ref,
                 h_ref, xs_ref):
    h, dis = _arma_h(raw0_ref, raw1_ref, xin_ref, degp_ref, w_ref, b_ref)
    h_ref[...] = h
    xs = h * dis
    xs_ref[0] = xs[:, :DH]
    xs_ref[1] = xs[:, DH:]


def _layer2_body(raw0_ref, raw1_ref, xin_ref, degp_ref, w_ref, b_ref,
                 fcw_ref, fcb_ref, out_ref):
    h, _ = _arma_h(raw0_ref, raw1_ref, xin_ref, degp_ref, w_ref, b_ref)
    out_ref[...] = (jnp.dot(h.astype(jnp.bfloat16), fcw_ref[...],
                            preferred_element_type=jnp.float32) + fcb_ref[...])


_degp_spec = pl.BlockSpec((NC, BLK, DEG_WOUT), lambda i: (0, i, 0))
_row_spec = pl.BlockSpec((BLK, D), lambda i: (i, 0))
_half0_spec = pl.BlockSpec((BLK, DH), lambda i: (i, 0))
_half1_spec = pl.BlockSpec((BLK, DH), lambda i: (i + NBLK, 0))
_split_spec = pl.BlockSpec((NC, BLK, DH), lambda i: (0, i, 0))

_scale_call = pl.pallas_call(
    _scale_kernel_body,
    grid=(NBLK,),
    in_specs=[_degp_spec, _row_spec],
    out_specs=_split_spec,
    out_shape=jax.ShapeDtypeStruct((NC, N, DH), jnp.float32),
)

_layer1_call = pl.pallas_call(
    _layer1_body,
    grid=(NBLK,),
    in_specs=[
        _half0_spec, _half1_spec, _row_spec, _degp_spec,
        pl.BlockSpec((2 * D, 2 * D), lambda i: (0, 0)),
        pl.BlockSpec((1, 2 * D), lambda i: (0, 0)),
    ],
    out_specs=[_row_spec, _split_spec],
    out_shape=[
        jax.ShapeDtypeStruct((N, D), jnp.float32),
        jax.ShapeDtypeStruct((NC, N, DH), jnp.float32),
    ],
)

_layer2_call = pl.pallas_call(
    _layer2_body,
    grid=(NBLK,),
    in_specs=[
        _half0_spec, _half1_spec, _row_spec, _degp_spec,
        pl.BlockSpec((2 * D, 2 * D), lambda i: (0, 0)),
        pl.BlockSpec((1, 2 * D), lambda i: (0, 0)),
        pl.BlockSpec((D, D), lambda i: (0, 0)),
        pl.BlockSpec((1, D), lambda i: (0, 0)),
    ],
    out_specs=_row_spec,
    out_shape=jax.ShapeDtypeStruct((N, D), jnp.float32),
)


def _wcat(iw, rw):
    return jnp.concatenate(
        [jnp.concatenate([iw[0], iw[1]], axis=1),
         jnp.concatenate([rw[0], rw[1]], axis=1)], axis=0)


def kernel(x, edge_index, init_w0, root_w0, bias0, init_w1, root_w1, bias1,
           fc_w, fc_b):
    src = edge_index[0]
    dst = edge_index[1]
    src2 = jnp.concatenate([src, src + N])
    zeros_hbm = jnp.zeros((ROWS_MAIN, DH), jnp.float32)
    ones_hbm = jnp.ones((DEG_CHUNK, DEG_W), jnp.float32)

    w0 = _wcat(init_w0, root_w0).astype(jnp.bfloat16)
    b0 = jnp.concatenate([bias0[0, 0], bias0[1, 0]])[None, :]
    w1 = _wcat(init_w1, root_w1).astype(jnp.bfloat16)
    b1 = jnp.concatenate([bias1[0, 0], bias1[1, 0]])[None, :]
    fcwT = fc_w.T.astype(jnp.bfloat16)
    fcb = fc_b[None, :]

    degp = _deg_kernel(dst, zeros_hbm, ones_hbm)[:, :, :DEG_WOUT]
    xs1 = _scale_call(degp, x)
    raw1 = _prop_kernel(xs1.reshape(NC * N, DH), src2, dst, zeros_hbm)
    h1, xs2 = _layer1_call(raw1, raw1, x, degp, w0, b0)
    raw2 = _prop_kernel(xs2.reshape(NC * N, DH), src2, dst, zeros_hbm)
    return _layer2_call(raw2, raw2, h1, degp, w1, b1, fcwT, fcb)


# speedup vs baseline: 1.0126x; 1.0126x over previous
import functools

import jax
import jax.numpy as jnp
from jax import lax
from jax.experimental import pallas as pl
from jax.experimental.pallas import tpu as pltpu
from jax.experimental.pallas import tpu_sc as plsc

N = 10000
E = 160000
D = 256
DH = 128
NC = 2
NS = 16
ROWS_MAIN = 632
ROWS_LAST = 520
BLK = 400
NBLK = N // BLK

DEG_EDGES_PER_W = E // (NC * NS)
DEG_CHUNK = 40
DEG_NCHUNK = DEG_EDGES_PER_W // DEG_CHUNK
DEG_NBUF = 5
DEG_OUTER = DEG_NCHUNK // DEG_NBUF
DEG_W = 128
DEG_WOUT = 16

PROP_EDGES_PER_SUB = E // NS
PROP_CHUNK = 40
PROP_NCHUNK = PROP_EDGES_PER_SUB // PROP_CHUNK
PROP_NBUF = 5
PROP_OUTER = PROP_NCHUNK // PROP_NBUF
PROP_EPI = PROP_NCHUNK - PROP_OUTER * PROP_NBUF

_MESH = plsc.VectorSubcoreMesh(core_axis_name="c", subcore_axis_name="s")


@functools.partial(
    pl.kernel,
    out_type=jax.ShapeDtypeStruct((NC, N, DEG_W), jnp.float32),
    mesh=_MESH,
    scratch_types=[
        pltpu.VMEM((DEG_EDGES_PER_W,), jnp.int32),
        pltpu.VMEM((DEG_CHUNK, DEG_W), jnp.float32),
        pltpu.VMEM_SHARED((N, DEG_W), jnp.float32),
        pltpu.SemaphoreType.DMA,
        pltpu.SemaphoreType.DMA,
        pltpu.SemaphoreType.DMA,
    ],
)
def _deg_kernel(dst_hbm, zeros_hbm, ones_hbm, degp_hbm, dst_all, ones_v,
                acc_sh, sem, psem0, psem1):
    c = lax.axis_index("c")
    s = lax.axis_index("s")
    w = c * NS + s
    pd = [
        pltpu.async_copy(ones_hbm, ones_v, psem0),
        pltpu.async_copy(
            dst_hbm.at[pl.ds(w * DEG_EDGES_PER_W, DEG_EDGES_PER_W)], dst_all,
            psem1),
    ]

    @pl.when(s < NS - 1)
    def _():
        pltpu.sync_copy(zeros_hbm.at[pl.ds(0, ROWS_MAIN), :],
                        acc_sh.at[pl.ds(s * ROWS_MAIN, ROWS_MAIN), :])

    @pl.when(s == NS - 1)
    def _():
        pltpu.sync_copy(zeros_hbm.at[pl.ds(0, ROWS_LAST), :],
                        acc_sh.at[pl.ds((NS - 1) * ROWS_MAIN, ROWS_LAST), :])

    for d in pd:
        d.wait()
    plsc.subcore_barrier()

    def body(t, carry):
        j0 = t * DEG_NBUF
        for b in range(DEG_NBUF):
            @pl.when(t > 0)
            def _():
                pltpu.make_async_copy(zeros_hbm.at[pl.ds(0, DEG_CHUNK), :],
                                      ones_v, sem).wait()
            idx = dst_all.at[pl.ds((j0 + b) * DEG_CHUNK, DEG_CHUNK)]
            pltpu.async_copy(ones_v, acc_sh.at[idx], sem, add=True)
        return carry

    lax.fori_loop(0, DEG_OUTER, body, 0)
    for b in range(DEG_NBUF):
        pltpu.make_async_copy(zeros_hbm.at[pl.ds(0, DEG_CHUNK), :],
                              ones_v, sem).wait()
    plsc.subcore_barrier()

    @pl.when(s < NS - 1)
    def _():
        pltpu.sync_copy(acc_sh.at[pl.ds(s * ROWS_MAIN, ROWS_MAIN), :],
                        degp_hbm.at[c, pl.ds(s * ROWS_MAIN, ROWS_MAIN), :])

    @pl.when(s == NS - 1)
    def _():
        pltpu.sync_copy(acc_sh.at[pl.ds((NS - 1) * ROWS_MAIN, ROWS_LAST), :],
                        degp_hbm.at[c, pl.ds((NS - 1) * ROWS_MAIN, ROWS_LAST), :])


@functools.partial(
    pl.kernel,
    out_type=jax.ShapeDtypeStruct((NC * N, DH), jnp.float32),
    mesh=_MESH,
    scratch_types=[
        pltpu.VMEM((PROP_EDGES_PER_SUB,), jnp.int32),
        pltpu.VMEM((PROP_EDGES_PER_SUB,), jnp.int32),
        [pltpu.VMEM((PROP_CHUNK, DH), jnp.float32) for _ in range(PROP_NBUF)],
        pltpu.VMEM_SHARED((N, DH), jnp.float32),
        [pltpu.SemaphoreType.DMA for _ in range(PROP_NBUF)],
        [pltpu.SemaphoreType.DMA for _ in range(PROP_NBUF)],
    ],
)
def _prop_kernel(xs_hbm, src2_hbm, dst_hbm, zeros_hbm, raw_hbm,
                 src_all, dst_all, bufs, acc_sh, gsems, ssems):
    c = lax.axis_index("c")
    s = lax.axis_index("s")

    pd = [
        pltpu.async_copy(src2_hbm.at[pl.ds(c * E + s * PROP_EDGES_PER_SUB,
                                           PROP_EDGES_PER_SUB)], src_all,
                         gsems[0]),
        pltpu.async_copy(dst_hbm.at[pl.ds(s * PROP_EDGES_PER_SUB,
                                          PROP_EDGES_PER_SUB)], dst_all,
                         gsems[1]),
    ]

    @pl.when(s < NS - 1)
    def _():
        pltpu.sync_copy(zeros_hbm.at[pl.ds(0, ROWS_MAIN), :],
                        acc_sh.at[pl.ds(s * ROWS_MAIN, ROWS_MAIN), :])

    @pl.when(s == NS - 1)
    def _():
        pltpu.sync_copy(zeros_hbm.at[pl.ds(0, ROWS_LAST), :],
                        acc_sh.at[pl.ds((NS - 1) * ROWS_MAIN, ROWS_LAST), :])

    for d in pd:
        d.wait()
    plsc.subcore_barrier()

    def body(t, carry):
        j0 = t * PROP_NBUF
        gd = []
        for b in range(PROP_NBUF):
            @pl.when(t > 0)
            def _(b=b):
                pltpu.make_async_copy(
                    zeros_hbm.at[pl.ds(0, PROP_CHUNK), :], bufs[b],
                    ssems[b]).wait()
            idx = src_all.at[pl.ds((j0 + b) * PROP_CHUNK, PROP_CHUNK)]
            gd.append(pltpu.async_copy(xs_hbm.at[idx], bufs[b], gsems[b]))
        for b in range(PROP_NBUF):
            gd[b].wait()
            idx = dst_all.at[pl.ds((j0 + b) * PROP_CHUNK, PROP_CHUNK)]
            pltpu.async_copy(bufs[b], acc_sh.at[idx], ssems[b], add=True)
        return carry

    lax.fori_loop(0, PROP_OUTER, body, 0)
    for b in range(PROP_NBUF):
        pltpu.make_async_copy(zeros_hbm.at[pl.ds(0, PROP_CHUNK), :], bufs[b],
                              ssems[b]).wait()
    plsc.subcore_barrier()

    @pl.when(s < NS - 1)
    def _():
        pltpu.sync_copy(acc_sh.at[pl.ds(s * ROWS_MAIN, ROWS_MAIN), :],
                        raw_hbm.at[pl.ds(c * N + s * ROWS_MAIN, ROWS_MAIN), :])

    @pl.when(s == NS - 1)
    def _():
        pltpu.sync_copy(
            acc_sh.at[pl.ds((NS - 1) * ROWS_MAIN, ROWS_LAST), :],
            raw_hbm.at[pl.ds(c * N + (NS - 1) * ROWS_MAIN, ROWS_LAST), :])


def _dis_from_degp(degp_blk):
    dv = degp_blk[0] + degp_blk[1]
    deg = dv[:, 0:1]
    return jnp.where(deg > 0, lax.rsqrt(deg), 0.0)


def _scale_kernel_body(degp_ref, x_ref, xs_ref):
    dis = _dis_from_degp(degp_ref[...])
    xs = x_ref[...] * dis
    xs_ref[0] = xs[:, :DH]
    xs_ref[1] = xs[:, DH:]


def _arma_h(raw0_ref, raw1_ref, xin_ref, degp_ref, w_ref, b_ref):
    dis = _dis_from_degp(degp_ref[...])
    agg = jnp.concatenate([raw0_ref[...], raw1_ref[...]], axis=1) * dis
    cat = jnp.concatenate([agg, xin_ref[...]], axis=1)
    z = jnp.dot(cat.astype(jnp.bfloat16), w_ref[...],
                preferred_element_type=jnp.float32)
    z = z + b_ref[...]
    h = 0.5 * (jnp.maximum(z[:, :D], 0.0) + jnp.maximum(z[:, D:], 0.0))
    return h, dis


def _layer1_body(raw0_ref, raw1_ref, xin_ref, degp_ref, w_ref, b_ref,
                 h_ref, xs_ref):
    h, dis = _arma_h(raw0_ref, raw1_ref, xin_ref, degp_ref, w_ref, b_ref)
    h_ref[...] = h
    xs = h * dis
    xs_ref[0] = xs[:, :DH]
    xs_ref[1] = xs[:, DH:]


def _layer2_body(raw0_ref, raw1_ref, xin_ref, degp_ref, w_ref, b_ref,
                 fcw_ref, fcb_ref, out_ref):
    h, _ = _arma_h(raw0_ref, raw1_ref, xin_ref, degp_ref, w_ref, b_ref)
    out_ref[...] = (jnp.dot(h.astype(jnp.bfloat16), fcw_ref[...],
                            preferred_element_type=jnp.float32) + fcb_ref[...])


_degp_spec = pl.BlockSpec((NC, BLK, DEG_WOUT), lambda i: (0, i, 0))
_row_spec = pl.BlockSpec((BLK, D), lambda i: (i, 0))
_half0_spec = pl.BlockSpec((BLK, DH), lambda i: (i, 0))
_half1_spec = pl.BlockSpec((BLK, DH), lambda i: (i + NBLK, 0))
_split_spec = pl.BlockSpec((NC, BLK, DH), lambda i: (0, i, 0))

_scale_call = pl.pallas_call(
    _scale_kernel_body,
    grid=(NBLK,),
    in_specs=[_degp_spec, _row_spec],
    out_specs=_split_spec,
    out_shape=jax.ShapeDtypeStruct((NC, N, DH), jnp.float32),
)

_layer1_call = pl.pallas_call(
    _layer1_body,
    grid=(NBLK,),
    in_specs=[
        _half0_spec, _half1_spec, _row_spec, _degp_spec,
        pl.BlockSpec((2 * D, 2 * D), lambda i: (0, 0)),
        pl.BlockSpec((1, 2 * D), lambda i: (0, 0)),
    ],
    out_specs=[_row_spec, _split_spec],
    out_shape=[
        jax.ShapeDtypeStruct((N, D), jnp.float32),
        jax.ShapeDtypeStruct((NC, N, DH), jnp.float32),
    ],
)

_layer2_call = pl.pallas_call(
    _layer2_body,
    grid=(NBLK,),
    in_specs=[
        _half0_spec, _half1_spec, _row_spec, _degp_spec,
        pl.BlockSpec((2 * D, 2 * D), lambda i: (0, 0)),
        pl.BlockSpec((1, 2 * D), lambda i: (0, 0)),
        pl.BlockSpec((D, D), lambda i: (0, 0)),
        pl.BlockSpec((1, D), lambda i: (0, 0)),
    ],
    out_specs=_row_spec,
    out_shape=jax.ShapeDtypeStruct((N, D), jnp.float32),
)


def _wcat(iw, rw):
    return jnp.concatenate(
        [jnp.concatenate([iw[0], iw[1]], axis=1),
         jnp.concatenate([rw[0], rw[1]], axis=1)], axis=0)


def kernel(x, edge_index, init_w0, root_w0, bias0, init_w1, root_w1, bias1,
           fc_w, fc_b):
    src = edge_index[0]
    dst = edge_index[1]
    src2 = jnp.concatenate([src, src + N])
    zeros_hbm = jnp.zeros((ROWS_MAIN, DH), jnp.float32)
    ones_hbm = jnp.ones((DEG_CHUNK, DEG_W), jnp.float32)

    w0 = _wcat(init_w0, root_w0).astype(jnp.bfloat16)
    b0 = jnp.concatenate([bias0[0, 0], bias0[1, 0]])[None, :]
    w1 = _wcat(init_w1, root_w1).astype(jnp.bfloat16)
    b1 = jnp.concatenate([bias1[0, 0], bias1[1, 0]])[None, :]
    fcwT = fc_w.T.astype(jnp.bfloat16)
    fcb = fc_b[None, :]

    degp = _deg_kernel(dst, zeros_hbm, ones_hbm)[:, :, :DEG_WOUT]
    xs1 = _scale_call(degp, x)
    raw1 = _prop_kernel(xs1.reshape(NC * N, DH), src2, dst, zeros_hbm)
    h1, xs2 = _layer1_call(raw1, raw1, x, degp, w0, b0)
    raw2 = _prop_kernel(xs2.reshape(NC * N, DH), src2, dst, zeros_hbm)
    return _layer2_call(raw2, raw2, h1, degp, w1, b1, fcwT, fcb)
